# baseline (speedup 1.0000x reference)
import jax
import jax.numpy as jnp
from jax import lax
from jax.experimental import pallas as pl
from jax.experimental.pallas import tpu as pltpu
from jax.experimental.pallas import tpu_sc as plsc

_B, _T, _D = 4096, 200, 64
_N = _B * _T
_CTX = 200
_NC, _NS = 2, 16
_NW = _NC * _NS
_BPW = _B // _NW
_L = 16
_SEG = (128, 72)


def _emb_body(pv_hbm, pi2_hbm, vt2, pos2, out_hbm,
              pvb, pi2b, bufs, sbufs, pos_sp, sem_gv, sem_ga, sem_st):
    wid = lax.axis_index("s") * _NC + lax.axis_index("c")
    wb = wid * _BPW
    base0 = wb * _T

    pl.when(lax.axis_index("s") == 0)(lambda: pltpu.sync_copy(pos2, pos_sp))
    plsc.subcore_barrier()

    iota = lax.iota(jnp.int32, _L)

    def do_row(slot, r):
        base = base0 + r * _T
        pltpu.sync_copy(pv_hbm.at[pl.ds(base, _T)],
                        pvb.at[pl.ds(slot * _T, _T)])
        pltpu.sync_copy(pi2_hbm.at[pl.ds(base, _T)],
                        pi2b.at[pl.ds(slot * _T, _T)])
        o = 0
        for seg in _SEG:
            pltpu.async_copy(
                vt2.at[pvb.at[pl.ds(slot * _T + o, seg)]],
                bufs.at[slot, pl.ds(o, seg)], sem_gv.at[slot])
            o += seg
        o = 0
        for seg in _SEG:
            pltpu.make_async_copy(
                vt2.at[pvb.at[pl.ds(slot * _T + o, seg)]],
                bufs.at[slot, pl.ds(o, seg)], sem_gv.at[slot]).wait()
            o += seg
        o = 0
        for seg in _SEG:
            pltpu.async_copy(
                pos_sp.at[pi2b.at[pl.ds(slot * _T + o, seg)]],
                bufs.at[slot, pl.ds(o, seg)], sem_ga.at[slot], add=True)
            o += seg
        o = 0
        for seg in _SEG:
            pltpu.make_async_copy(
                pos_sp.at[pi2b.at[pl.ds(slot * _T + o, seg)]],
                bufs.at[slot, pl.ds(o, seg)], sem_ga.at[slot]).wait()
            o += seg

        def extract(j, carry):
            start = lax.min(16 * j, _T - _L)
            rows = start + iota
            i2 = pi2b[pl.ds(slot * _T + start, _L)]
            col0 = (i2 & 1) * _D
            for d in range(_D):
                v = plsc.load_gather(bufs.at[slot], [rows, col0 + d])
                plsc.store_scatter(
                    sbufs.at[slot], [rows, jnp.full((_L,), d, jnp.int32)], v)
            return carry

        lax.fori_loop(0, (_T + _L - 1) // _L, extract, 0)
        pltpu.sync_copy(sbufs.at[slot], out_hbm.at[wb + r])

    def body(i, carry):
        do_row(0, 2 * i)
        do_row(1, 2 * i + 1)
        return carry

    lax.fori_loop(0, _BPW // 2, body, 0)


@jax.jit
def _emb(pv, pi2, vt2, pos2):
    f = pl.kernel(
        _emb_body,
        out_type=jax.ShapeDtypeStruct((_B, _T, _D), jnp.float32),
        mesh=plsc.VectorSubcoreMesh(
            core_axis_name="c", subcore_axis_name="s",
            num_cores=_NC, num_subcores=_NS),
        scratch_types=[
            pltpu.VMEM((2 * _T,), jnp.int32),
            pltpu.VMEM((2 * _T,), jnp.int32),
            pltpu.VMEM((2, _T, 2 * _D), jnp.float32),
            pltpu.VMEM((2, _T, _D), jnp.float32),
            pltpu.VMEM_SHARED((2 * _CTX, 2 * _D), jnp.float32),
            pltpu.SemaphoreType.DMA((2,)),
            pltpu.SemaphoreType.DMA((2,)),
            pltpu.SemaphoreType.DMA((2,)),
        ],
        compiler_params=pltpu.CompilerParams(needs_layout_passes=False),
    )
    return f(pv, pi2, vt2, pos2)


def kernel(x, pos_idx, value_table, pos_table):
    xf = x.reshape(_N)
    pf = pos_idx.reshape(_N)
    pv = xf >> 1
    pi2 = 2 * pf + (xf & 1)
    vt2 = value_table.reshape(500000, 2 * _D)
    z = jnp.zeros((_CTX, _D), jnp.float32)
    even = jnp.concatenate([pos_table, z], axis=1)
    odd = jnp.concatenate([z, pos_table], axis=1)
    pos2 = jnp.stack([even, odd], axis=1).reshape(2 * _CTX, 2 * _D)
    return _emb(pv, pi2, vt2, pos2)


# speedup vs baseline: 2.3936x; 2.3936x over previous
import jax
import jax.numpy as jnp
from jax import lax
from jax.experimental import pallas as pl
from jax.experimental.pallas import tpu as pltpu
from jax.experimental.pallas import tpu_sc as plsc

_B, _T, _D = 4096, 200, 64
_N = _B * _T
_CTX = 200
_NC, _NS = 2, 16
_NW = _NC * _NS
_RPW = _N // _NW
_C = 64
_U = 4
_GROUP = 2 * _U * _C
_NBODY = _RPW // _GROUP


def _emb_body(x_hbm, pi_hbm, val_tab, pos_tab, out_hbm,
              xi_all, pi_all, pos_vt, bufs, sem_gv, sem_ga, sem_st):
    wid = lax.axis_index("s") * _NC + lax.axis_index("c")
    base0 = wid * _RPW

    pltpu.sync_copy(x_hbm.at[pl.ds(base0, _RPW)], xi_all)
    pltpu.sync_copy(pi_hbm.at[pl.ds(base0, _RPW)], pi_all)
    pl.when(lax.axis_index("s") == 0)(lambda: pltpu.sync_copy(pos_tab, pos_vt))
    plsc.subcore_barrier()

    def off(k, h, u):
        return k * _GROUP + h * (_U * _C) + u * _C

    def gv(k, h, u):
        pltpu.async_copy(
            val_tab.at[xi_all.at[pl.ds(off(k, h, u), _C)]],
            bufs.at[h, u], sem_gv.at[h, u])

    def gv_wait(h, u):
        pltpu.make_async_copy(
            val_tab.at[xi_all.at[pl.ds(0, _C)]],
            bufs.at[h, u], sem_gv.at[h, u]).wait()

    def ga(k, h, u):
        pltpu.async_copy(
            pos_vt.at[pi_all.at[pl.ds(off(k, h, u), _C)]],
            bufs.at[h, u], sem_ga.at[h, u], add=True)

    def ga_wait(h, u):
        pltpu.make_async_copy(
            pos_vt.at[pi_all.at[pl.ds(0, _C)]],
            bufs.at[h, u], sem_ga.at[h, u]).wait()

    def st(k, h, u):
        pltpu.async_copy(
            bufs.at[h, u],
            out_hbm.at[pl.ds(base0 + off(k, h, u), _C)],
            sem_st.at[h, u])

    def st_wait(h, u):
        pltpu.make_async_copy(
            bufs.at[h, u],
            out_hbm.at[pl.ds(base0, _C)], sem_st.at[h, u]).wait()

    for u in range(_U):
        gv(0, 0, u)

    def body(k, carry):
        for u in range(_U):
            gv_wait(0, u)
            ga(k, 0, u)
        for u in range(_U):
            pl.when(k > 0)(lambda u=u: st_wait(1, u))
            gv(k, 1, u)
        for u in range(_U):
            ga_wait(0, u)
            st(k, 0, u)
        for u in range(_U):
            gv_wait(1, u)
            ga(k, 1, u)
        for u in range(_U):
            st_wait(0, u)
            pl.when(k < _NBODY - 1)(lambda u=u: gv(k + 1, 0, u))
        for u in range(_U):
            ga_wait(1, u)
            st(k, 1, u)
        return carry

    lax.fori_loop(0, _NBODY, body, 0)

    for u in range(_U):
        st_wait(1, u)


_V = 1000000
_DC = 160
_DNC = _V // _DC
_DNJ = (_DNC + _NW - 1) // _NW
_DNJ += _DNJ % 2


def _depad_body(vt, out2, gbufs, pbufs, sem_ld, sem_st):
    wid = lax.axis_index("s") * _NC + lax.axis_index("c")

    def cidx(j):
        return lax.min(wid + _NW * j, _DNC - 1)

    def ld(j, s):
        pltpu.async_copy(
            vt.at[pl.ds(pl.multiple_of(cidx(j) * _DC, 8), _DC)],
            gbufs.at[s], sem_ld.at[s])

    def ld_wait(s):
        pltpu.make_async_copy(vt.at[pl.ds(0, _DC)],
                              gbufs.at[s], sem_ld.at[s]).wait()

    def st(j, s):
        pltpu.async_copy(
            pbufs.at[s],
            out2.at[pl.ds(pl.multiple_of(cidx(j) * (_DC // 2), 8),
                          _DC // 2)],
            sem_st.at[s])

    def st_wait(s):
        pltpu.make_async_copy(pbufs.at[s],
                              out2.at[pl.ds(0, _DC // 2)],
                              sem_st.at[s]).wait()

    def repack(s):
        def rowpair(rr, carry):
            for half in range(2):
                for c4 in range(_D // 16):
                    v = gbufs[s, 2 * rr + half, pl.ds(c4 * 16, 16)]
                    pbufs[s, rr, pl.ds(half * _D + c4 * 16, 16)] = v
            return carry
        lax.fori_loop(0, _DC // 2, rowpair, 0)

    for s in range(2):
        ld(0, s)

    def body(t, carry):
        for s in range(2):
            j = 2 * t + s
            ld_wait(s)
            pl.when(t > 0)(lambda s=s: st_wait(s))
            repack(s)
            st(j, s)
            pl.when(t < _DNJ // 2 - 1)(lambda j=j, s=s: ld(j + 2, s))
        return carry

    lax.fori_loop(0, _DNJ // 2, body, 0)
    for s in range(2):
        st_wait(s)


@jax.jit
def _emb(xf, pf, value_table, pos_table):
    fa = pl.kernel(
        _depad_body,
        out_type=jax.ShapeDtypeStruct((_V // 2, 2 * _D), jnp.float32),
        mesh=plsc.VectorSubcoreMesh(
            core_axis_name="c", subcore_axis_name="s",
            num_cores=_NC, num_subcores=_NS),
        scratch_types=[
            pltpu.VMEM((2, _DC, _D), jnp.float32),
            pltpu.VMEM((2, _DC // 2, 2 * _D), jnp.float32),
            pltpu.SemaphoreType.DMA((2,)),
            pltpu.SemaphoreType.DMA((2,)),
        ],
        compiler_params=pltpu.CompilerParams(needs_layout_passes=False),
    )
    value_table = fa(value_table).reshape(_V, _D)
    f = pl.kernel(
        _emb_body,
        out_type=jax.ShapeDtypeStruct((_N, _D), jnp.float32),
        mesh=plsc.VectorSubcoreMesh(
            core_axis_name="c", subcore_axis_name="s",
            num_cores=_NC, num_subcores=_NS),
        scratch_types=[
            pltpu.VMEM((_RPW,), jnp.int32),
            pltpu.VMEM((_RPW,), jnp.int32),
            pltpu.VMEM_SHARED((_CTX, _D), jnp.float32),
            pltpu.VMEM((2, _U, _C, _D), jnp.float32),
            pltpu.SemaphoreType.DMA((2, _U)),
            pltpu.SemaphoreType.DMA((2, _U)),
            pltpu.SemaphoreType.DMA((2, _U)),
        ],
        compiler_params=pltpu.CompilerParams(
            use_tc_tiling_on_sc=False, skip_device_barrier=True),
    )
    return f(xf, pf, value_table, pos_table)


def kernel(x, pos_idx, value_table, pos_table):
    xf = x.reshape(_N)
    pf = pos_idx.reshape(_N)
    out = _emb(xf, pf, value_table, pos_table)
    return out.reshape(_B, _T, _D)


# speedup vs baseline: 2.7470x; 1.1477x over previous
import jax
import jax.numpy as jnp
from jax import lax
from jax.experimental import pallas as pl
from jax.experimental.pallas import tpu as pltpu
from jax.experimental.pallas import tpu_sc as plsc

_B, _T, _D = 4096, 200, 64
_N = _B * _T
_CTX = 200
_NC, _NS = 2, 16
_NW = _NC * _NS
_RPW = _N // _NW
_C = 64
_U = 4
_GROUP = 2 * _U * _C
_NBODY = _RPW // _GROUP


def _emb_body(x_hbm, pi_hbm, val_tab, pos_tab, out_hbm,
              xi_all, pi_all, pos_vt, bufs, sem_gv, sem_ga, sem_st):
    wid = lax.axis_index("s") * _NC + lax.axis_index("c")
    base0 = wid * _RPW

    pltpu.sync_copy(x_hbm.at[pl.ds(base0, _RPW)], xi_all)
    pltpu.sync_copy(pi_hbm.at[pl.ds(base0, _RPW)], pi_all)
    pl.when(lax.axis_index("s") == 0)(lambda: pltpu.sync_copy(pos_tab, pos_vt))
    plsc.subcore_barrier()

    def off(k, h, u):
        return k * _GROUP + h * (_U * _C) + u * _C

    def gv(k, h, u):
        pltpu.async_copy(
            val_tab.at[xi_all.at[pl.ds(off(k, h, u), _C)]],
            bufs.at[h, u], sem_gv.at[h, u])

    def gv_wait(h, u):
        pltpu.make_async_copy(
            val_tab.at[xi_all.at[pl.ds(0, _C)]],
            bufs.at[h, u], sem_gv.at[h, u]).wait()

    def ga(k, h, u):
        pltpu.async_copy(
            pos_vt.at[pi_all.at[pl.ds(off(k, h, u), _C)]],
            bufs.at[h, u], sem_ga.at[h, u], add=True)

    def ga_wait(h, u):
        pltpu.make_async_copy(
            pos_vt.at[pi_all.at[pl.ds(0, _C)]],
            bufs.at[h, u], sem_ga.at[h, u]).wait()

    def st(k, h, u):
        pltpu.async_copy(
            bufs.at[h, u],
            out_hbm.at[pl.ds(base0 + off(k, h, u), _C)],
            sem_st.at[h, u])

    def st_wait(h, u):
        pltpu.make_async_copy(
            bufs.at[h, u],
            out_hbm.at[pl.ds(base0, _C)], sem_st.at[h, u]).wait()

    for u in range(_U):
        gv(0, 0, u)

    def body(k, carry):
        for u in range(_U):
            gv_wait(0, u)
            ga(k, 0, u)
        for u in range(_U):
            pl.when(k > 0)(lambda u=u: st_wait(1, u))
            gv(k, 1, u)
        for u in range(_U):
            ga_wait(0, u)
            st(k, 0, u)
        for u in range(_U):
            gv_wait(1, u)
            ga(k, 1, u)
        for u in range(_U):
            st_wait(0, u)
            pl.when(k < _NBODY - 1)(lambda u=u: gv(k + 1, 0, u))
        for u in range(_U):
            ga_wait(1, u)
            st(k, 1, u)
        return carry

    lax.fori_loop(0, _NBODY, body, 0)

    for u in range(_U):
        st_wait(1, u)


@jax.jit
def _emb(xf, pf, value_table, pos_table):
    f = pl.kernel(
        _emb_body,
        out_type=jax.ShapeDtypeStruct((_N, _D), jnp.float32),
        mesh=plsc.VectorSubcoreMesh(
            core_axis_name="c", subcore_axis_name="s",
            num_cores=_NC, num_subcores=_NS),
        scratch_types=[
            pltpu.VMEM((_RPW,), jnp.int32),
            pltpu.VMEM((_RPW,), jnp.int32),
            pltpu.VMEM_SHARED((_CTX, _D), jnp.float32),
            pltpu.VMEM((2, _U, _C, _D), jnp.float32),
            pltpu.SemaphoreType.DMA((2, _U)),
            pltpu.SemaphoreType.DMA((2, _U)),
            pltpu.SemaphoreType.DMA((2, _U)),
        ],
        compiler_params=pltpu.CompilerParams(
            use_tc_tiling_on_sc=False, skip_device_barrier=True),
    )
    return f(xf, pf, value_table, pos_table)


def kernel(x, pos_idx, value_table, pos_table):
    xf = x.reshape(_N)
    pf = pos_idx.reshape(_N)
    out = _emb(xf, pf, value_table, pos_table)
    return out.reshape(_B, _T, _D)


# speedup vs baseline: 2.7835x; 1.0133x over previous
import jax
import jax.numpy as jnp
from jax import lax
from jax.experimental import pallas as pl
from jax.experimental.pallas import tpu as pltpu
from jax.experimental.pallas import tpu_sc as plsc

_B, _T, _D = 4096, 200, 64
_N = _B * _T
_CTX = 200
_NC, _NS = 2, 16
_NW = _NC * _NS
_RPW = _N // _NW
_C = 128
_U = 4
_GROUP = 2 * _U * _C
_NBODY = _RPW // _GROUP


def _emb_body(x_hbm, pi_hbm, val_tab, pos_tab, out_hbm,
              xi_all, pi_all, pos_vt, bufs, sem_gv, sem_ga, sem_st):
    wid = lax.axis_index("s") * _NC + lax.axis_index("c")
    base0 = wid * _RPW

    pltpu.sync_copy(x_hbm.at[pl.ds(base0, _RPW)], xi_all)
    pltpu.sync_copy(pi_hbm.at[pl.ds(base0, _RPW)], pi_all)
    pl.when(lax.axis_index("s") == 0)(lambda: pltpu.sync_copy(pos_tab, pos_vt))
    plsc.subcore_barrier()

    def off(k, h, u):
        return k * _GROUP + h * (_U * _C) + u * _C

    def gv(k, h, u):
        pltpu.async_copy(
            val_tab.at[xi_all.at[pl.ds(off(k, h, u), _C)]],
            bufs.at[h, u], sem_gv.at[h, u])

    def gv_wait(h, u):
        pltpu.make_async_copy(
            val_tab.at[xi_all.at[pl.ds(0, _C)]],
            bufs.at[h, u], sem_gv.at[h, u]).wait()

    def ga(k, h, u):
        pltpu.async_copy(
            pos_vt.at[pi_all.at[pl.ds(off(k, h, u), _C)]],
            bufs.at[h, u], sem_ga.at[h, u], add=True)

    def ga_wait(h, u):
        pltpu.make_async_copy(
            pos_vt.at[pi_all.at[pl.ds(0, _C)]],
            bufs.at[h, u], sem_ga.at[h, u]).wait()

    def st(k, h, u):
        pltpu.async_copy(
            bufs.at[h, u],
            out_hbm.at[pl.ds(base0 + off(k, h, u), _C)],
            sem_st.at[h, u])

    def st_wait(h, u):
        pltpu.make_async_copy(
            bufs.at[h, u],
            out_hbm.at[pl.ds(base0, _C)], sem_st.at[h, u]).wait()

    for u in range(_U):
        gv(0, 0, u)

    def body(k, carry):
        for u in range(_U):
            gv_wait(0, u)
            ga(k, 0, u)
        for u in range(_U):
            pl.when(k > 0)(lambda u=u: st_wait(1, u))
            gv(k, 1, u)
        for u in range(_U):
            ga_wait(0, u)
            st(k, 0, u)
        for u in range(_U):
            gv_wait(1, u)
            ga(k, 1, u)
        for u in range(_U):
            st_wait(0, u)
            pl.when(k < _NBODY - 1)(lambda u=u: gv(k + 1, 0, u))
        for u in range(_U):
            ga_wait(1, u)
            st(k, 1, u)
        return carry

    lax.fori_loop(0, _NBODY, body, 0)

    for u in range(_U):
        st_wait(1, u)


@jax.jit
def _emb(xf, pf, value_table, pos_table):
    f = pl.kernel(
        _emb_body,
        out_type=jax.ShapeDtypeStruct((_N, _D), jnp.float32),
        mesh=plsc.VectorSubcoreMesh(
            core_axis_name="c", subcore_axis_name="s",
            num_cores=_NC, num_subcores=_NS),
        scratch_types=[
            pltpu.VMEM((_RPW,), jnp.int32),
            pltpu.VMEM((_RPW,), jnp.int32),
            pltpu.VMEM_SHARED((_CTX, _D), jnp.float32),
            pltpu.VMEM((2, _U, _C, _D), jnp.float32),
            pltpu.SemaphoreType.DMA((2, _U)),
            pltpu.SemaphoreType.DMA((2, _U)),
            pltpu.SemaphoreType.DMA((2, _U)),
        ],
        compiler_params=pltpu.CompilerParams(
            use_tc_tiling_on_sc=False, skip_device_barrier=True),
    )
    return f(xf, pf, value_table, pos_table)


def kernel(x, pos_idx, value_table, pos_table):
    xf = x.reshape(_N)
    pf = pos_idx.reshape(_N)
    out = _emb(xf, pf, value_table, pos_table)
    return out.reshape(_B, _T, _D)
